# baseline (speedup 1.0000x reference)
import jax
import jax.numpy as jnp
from jax.experimental import pallas as pl

_BLOCK = 2048


def _innernode_tc_kernel(x_ref, w_ref, b_ref, o_ref):
    x = x_ref[...]
    w = w_ref[...]
    logits = jax.lax.dot_general(
        x, w, (((1,), (0,)), ((), ())),
        preferred_element_type=jnp.float32)
    mask = (logits[:, 0] + b_ref[0]) > 0.0
    o_ref[...] = jnp.where(mask, 0, 1).astype(o_ref.dtype)


def kernel(feat0, feat1, feat2, w, b):
    del feat0, feat2
    N, d = feat1.shape
    out_dtype = jnp.zeros((), dtype=jnp.int64).dtype
    w2 = w.reshape(d, 1)
    b1 = b.reshape(1)
    grid = (N // _BLOCK,)
    return pl.pallas_call(
        _innernode_tc_kernel,
        grid=grid,
        in_specs=[
            pl.BlockSpec((_BLOCK, d), lambda i: (i, 0)),
            pl.BlockSpec((d, 1), lambda i: (0, 0)),
            pl.BlockSpec((1,), lambda i: (0,)),
        ],
        out_specs=pl.BlockSpec((_BLOCK,), lambda i: (i,)),
        out_shape=jax.ShapeDtypeStruct((N,), out_dtype),
    )(feat1, w2, b1)


# speedup vs baseline: 1.0237x; 1.0237x over previous
import jax
import jax.numpy as jnp
from jax.experimental import pallas as pl

_BLOCK = 4096


def _innernode_tc_kernel(x_ref, w_ref, b_ref, o_ref):
    x = x_ref[...]
    w = w_ref[...]
    logits = jax.lax.dot_general(
        x, w, (((1,), (0,)), ((), ())),
        preferred_element_type=jnp.float32)
    mask = (logits[:, 0] + b_ref[0]) > 0.0
    o_ref[...] = jnp.where(mask, 0, 1).astype(o_ref.dtype)


def kernel(feat0, feat1, feat2, w, b):
    del feat0, feat2
    N, d = feat1.shape
    out_dtype = jnp.zeros((), dtype=jnp.int64).dtype
    w2 = w.reshape(d, 1)
    b1 = b.reshape(1)
    grid = (N // _BLOCK,)
    return pl.pallas_call(
        _innernode_tc_kernel,
        grid=grid,
        in_specs=[
            pl.BlockSpec((_BLOCK, d), lambda i: (i, 0)),
            pl.BlockSpec((d, 1), lambda i: (0, 0)),
            pl.BlockSpec((1,), lambda i: (0,)),
        ],
        out_specs=pl.BlockSpec((_BLOCK,), lambda i: (i,)),
        out_shape=jax.ShapeDtypeStruct((N,), out_dtype),
    )(feat1, w2, b1)


# speedup vs baseline: 1.1050x; 1.0794x over previous
import jax
import jax.numpy as jnp
from jax.experimental import pallas as pl

_BLOCK = 4096


def _innernode_tc_kernel(x_ref, w_ref, b_ref, o_ref):
    x = x_ref[...]
    w = w_ref[...]
    logits = jnp.sum(x * w, axis=1)
    mask = (logits + b_ref[0]) > 0.0
    o_ref[...] = jnp.where(mask, 0, 1).astype(o_ref.dtype)


def kernel(feat0, feat1, feat2, w, b):
    del feat0, feat2
    N, d = feat1.shape
    out_dtype = jnp.zeros((), dtype=jnp.int64).dtype
    w2 = w.reshape(1, d)
    b1 = b.reshape(1)
    grid = (N // _BLOCK,)
    return pl.pallas_call(
        _innernode_tc_kernel,
        grid=grid,
        in_specs=[
            pl.BlockSpec((_BLOCK, d), lambda i: (i, 0)),
            pl.BlockSpec((1, d), lambda i: (0, 0)),
            pl.BlockSpec((1,), lambda i: (0,)),
        ],
        out_specs=pl.BlockSpec((_BLOCK,), lambda i: (i,)),
        out_shape=jax.ShapeDtypeStruct((N,), out_dtype),
    )(feat1, w2, b1)
